# Initial kernel scaffold; baseline (speedup 1.0000x reference)
import jax
import jax.numpy as jnp
from jax.experimental import pallas as pl


def kernel(x, adj, weight, bias):
    raise NotImplementedError("write your pallas kernel here")


# speedup vs baseline: 8.9057x; 8.9057x over previous
import jax
import jax.numpy as jnp
from jax.experimental import pallas as pl

N = 4096
D = 128
BI = 128
BJ = 512
EPS = 1e-15
MAXNORM = 1.0 - 1e-5
_HI = jax.lax.Precision.HIGHEST


def _mmp_kernel(xi_ref, x_ref, adj_ref, w_ref, b_ref, out_ref):
    i = pl.program_id(0)
    xi = xi_ref[...]
    pn2 = jnp.sum(xi * xi, axis=1, keepdims=True)
    beta = 1.0 - pn2

    def body(j, carry):
        acc_a, deg, acc_q = carry
        xj = x_ref[pl.ds(j * BJ, BJ), :]
        adjc = adj_ref[:, pl.ds(j * BJ, BJ)]
        qn2 = jnp.sum(xj * xj, axis=1)[None, :]
        g = jax.lax.dot_general(xi, xj, (((1,), (1,)), ((), ())),
                                precision=_HI,
                                preferred_element_type=jnp.float32)
        den = (1.0 - 2.0 * g) + pn2 * qn2
        na = den + qn2 * beta
        bg = beta * g
        numsq = (na * na) * pn2 - 2.0 * na * bg + (beta * beta) * qn2
        numsq = jnp.maximum(numsq, 0.0)
        invden = 1.0 / den
        subsq = numsq * (invden * invden) + EPS
        rsub = jax.lax.rsqrt(subsq)
        subn = subsq * rsub
        t = jnp.clip(subn, EPS, MAXNORM)
        atanh = 0.5 * jnp.log((1.0 + t) / (1.0 - t))
        phi = atanh * invden * rsub
        rows = i * BI + jax.lax.broadcasted_iota(jnp.int32, (BI, BJ), 0)
        cols = j * BJ + jax.lax.broadcasted_iota(jnp.int32, (BI, BJ), 1)
        wphi = jnp.where(rows == cols, 0.0, adjc * phi)
        acc_a = acc_a - jnp.sum(wphi * na, axis=1, keepdims=True)
        deg = deg + jnp.sum(adjc, axis=1, keepdims=True)
        acc_q = acc_q + jax.lax.dot_general(
            wphi, xj, (((1,), (0,)), ((), ())),
            precision=_HI, preferred_element_type=jnp.float32)
        return acc_a, deg, acc_q

    zcol = jnp.zeros((BI, 1), jnp.float32)
    acc_a, deg, acc_q = jax.lax.fori_loop(
        0, N // BJ, body, (zcol, zcol, jnp.zeros((BI, D), jnp.float32)))

    degc = jnp.maximum(deg, 1e-8)
    betam = jnp.maximum(beta, EPS)
    mean_t = (betam / degc) * (acc_a * xi + beta * acc_q)
    v = jax.lax.dot_general(mean_t, w_ref[...], (((1,), (0,)), ((), ())),
                            precision=_HI,
                            preferred_element_type=jnp.float32) + b_ref[...]
    v_norm = jnp.sqrt(jnp.sum(v * v, axis=1, keepdims=True) + EPS)
    second = jnp.tanh(jnp.clip(v_norm / betam, -15.0, 15.0)) * v / v_norm
    b2 = jnp.sum(second * second, axis=1, keepdims=True)
    ab = jnp.sum(xi * second, axis=1, keepdims=True)
    num = (1.0 + 2.0 * ab + b2) * xi + beta * second
    dn = 1.0 + 2.0 * ab + pn2 * b2
    res = num / jnp.maximum(dn, EPS)
    rn = jnp.sqrt(jnp.sum(res * res, axis=1, keepdims=True) + EPS)
    res = jnp.where(rn > MAXNORM, res / rn * MAXNORM, res)
    xin = jnp.sqrt(pn2 + EPS)
    fb = jnp.where(xin > MAXNORM, xi / xin * MAXNORM, xi)
    out_ref[...] = jnp.where(deg > 0.5, res, fb)


def kernel(x, adj, weight, bias):
    return pl.pallas_call(
        _mmp_kernel,
        grid=(N // BI,),
        in_specs=[
            pl.BlockSpec((BI, D), lambda i: (i, 0)),
            pl.BlockSpec((N, D), lambda i: (0, 0)),
            pl.BlockSpec((BI, N), lambda i: (i, 0)),
            pl.BlockSpec((D, D), lambda i: (0, 0)),
            pl.BlockSpec((1, D), lambda i: (0, 0)),
        ],
        out_specs=pl.BlockSpec((BI, D), lambda i: (i, 0)),
        out_shape=jax.ShapeDtypeStruct((N, D), jnp.float32),
    )(x, x, adj, weight, bias.reshape(1, D))


# speedup vs baseline: 37.3751x; 4.1968x over previous
import jax
import jax.numpy as jnp
from jax.experimental import pallas as pl

N = 4096
D = 128
BI = 128
BJ = 512
DA = 136
EPS = 1e-15
MAXNORM = 1.0 - 1e-5
_HI = jax.lax.Precision.HIGHEST


def _dot(a, b, dims):
    return jax.lax.dot_general(a, b, (dims, ((), ())), precision=_HI,
                               preferred_element_type=jnp.float32)


def _mmp_kernel(xi_ref, xa_ref, bden_ref, bpmq_ref, adj_ref, w_ref, b_ref,
                out_ref):
    i = pl.program_id(0)
    xi = xi_ref[...]
    xa = xa_ref[...]
    pn2 = jnp.sum(xi * xi, axis=1, keepdims=True)
    beta = 1.0 - pn2
    dchunk = (i * BI) // BJ

    def body(j, carry):
        a1v, degv, acc = carry
        bden = bden_ref[pl.ds(j * BJ, BJ), :]
        bpmq = bpmq_ref[pl.ds(j * BJ, BJ), :]
        adjc = adj_ref[:, pl.ds(j * BJ, BJ)]
        den = _dot(xa, bden, ((1,), (1,)))
        pmq2 = _dot(xa, bpmq, ((1,), (1,)))
        invden = 1.0 / jnp.maximum(den, 1e-9)
        subsq = jnp.maximum(pmq2, 0.0) * invden + EPS
        rsub = jax.lax.rsqrt(subsq)
        subn = subsq * rsub
        t = jnp.minimum(subn, MAXNORM)
        atanh = 0.5 * jnp.log((1.0 + t) / (1.0 - t))
        psi = atanh * rsub

        def masked(a, p):
            rows = i * BI + jax.lax.broadcasted_iota(jnp.int32, (BI, BJ), 0)
            cols = j * BJ + jax.lax.broadcasted_iota(jnp.int32, (BI, BJ), 1)
            return jnp.where(rows == cols, 0.0, a * p)

        wpsi = jax.lax.cond(j == dchunk, masked, lambda a, p: a * p,
                            adjc, psi)
        wphi = wpsi * invden
        a1v = a1v + wpsi
        degv = degv + adjc
        acc = acc + _dot(wphi, bden, ((1,), (0,)))
        return a1v, degv, acc

    zv = jnp.zeros((BI, BJ), jnp.float32)
    a1v, degv, acc = jax.lax.fori_loop(
        0, N // BJ, body, (zv, zv, jnp.zeros((BI, DA), jnp.float32)))

    a1 = jnp.sum(a1v, axis=1, keepdims=True)
    deg = jnp.sum(degv, axis=1, keepdims=True)
    a2 = acc[:, D:D + 1]
    acc_q = acc[:, :D]
    a_coef = -(a1 + beta * a2)
    degc = jnp.maximum(deg, 1e-8)
    betam = jnp.maximum(beta, EPS)
    mean_t = (betam / degc) * (a_coef * xi + beta * acc_q)
    v = _dot(mean_t, w_ref[...], ((1,), (0,))) + b_ref[...]
    v_norm = jnp.sqrt(jnp.sum(v * v, axis=1, keepdims=True) + EPS)
    second = jnp.tanh(jnp.clip(v_norm / betam, -15.0, 15.0)) * v / v_norm
    b2 = jnp.sum(second * second, axis=1, keepdims=True)
    ab = jnp.sum(xi * second, axis=1, keepdims=True)
    num = (1.0 + 2.0 * ab + b2) * xi + beta * second
    dn = 1.0 + 2.0 * ab + pn2 * b2
    res = num / jnp.maximum(dn, EPS)
    rn = jnp.sqrt(jnp.sum(res * res, axis=1, keepdims=True) + EPS)
    res = jnp.where(rn > MAXNORM, res / rn * MAXNORM, res)
    xin = jnp.sqrt(pn2 + EPS)
    fb = jnp.where(xin > MAXNORM, xi / xin * MAXNORM, xi)
    out_ref[...] = jnp.where(deg > 0.5, res, fb)


def kernel(x, adj, weight, bias):
    qn2 = jnp.sum(x * x, axis=1, keepdims=True)
    ones = jnp.ones((N, 1), jnp.float32)
    pad = jnp.zeros((N, DA - D - 2), jnp.float32)
    xa = jnp.concatenate([-2.0 * x, qn2, ones, pad], axis=1)
    bden = jnp.concatenate([x, qn2, ones, pad], axis=1)
    bpmq = jnp.concatenate([x, ones, qn2, pad], axis=1)
    return pl.pallas_call(
        _mmp_kernel,
        grid=(N // BI,),
        in_specs=[
            pl.BlockSpec((BI, D), lambda i: (i, 0)),
            pl.BlockSpec((BI, DA), lambda i: (i, 0)),
            pl.BlockSpec((N, DA), lambda i: (0, 0)),
            pl.BlockSpec((N, DA), lambda i: (0, 0)),
            pl.BlockSpec((BI, N), lambda i: (i, 0)),
            pl.BlockSpec((D, D), lambda i: (0, 0)),
            pl.BlockSpec((1, D), lambda i: (0, 0)),
        ],
        out_specs=pl.BlockSpec((BI, D), lambda i: (i, 0)),
        out_shape=jax.ShapeDtypeStruct((N, D), jnp.float32),
    )(x, xa, bden, bpmq, adj, weight, bias.reshape(1, D))


# speedup vs baseline: 48.1209x; 1.2875x over previous
import jax
import jax.numpy as jnp
from jax.experimental import pallas as pl

N = 4096
D = 128
BI = 128
BJ = 512
DA = 136
EPS = 1e-15
MAXNORM = 1.0 - 1e-5
_HI = jax.lax.Precision.HIGHEST


def _dot(a, b, dims):
    return jax.lax.dot_general(a, b, (dims, ((), ())), precision=_HI,
                               preferred_element_type=jnp.float32)


def _mmp_kernel(xi_ref, xa_ref, b_ref, adj_ref, w_ref, bias_ref, out_ref):
    i = pl.program_id(0)
    xi = xi_ref[...]
    xa = xa_ref[...]
    pn2 = jnp.sum(xi * xi, axis=1, keepdims=True)
    beta = 1.0 - pn2
    dchunk = (i * BI) // BJ

    def body(j, carry):
        a1v, degv, acc = carry
        bc = b_ref[pl.ds(j * BJ, BJ), :]
        adjc = adj_ref[:, pl.ds(j * BJ, BJ)]
        ehat = _dot(xa, bc, ((1,), (1,)))
        ehat = jnp.maximum(ehat, 0.0)
        r = 1.0 / (1.0 + ehat)
        ssq = (1.0 - r) + EPS
        rsub = jax.lax.rsqrt(ssq)
        subn = ssq * rsub
        t = jnp.minimum(subn, MAXNORM)
        atanh = 0.5 * jnp.log((1.0 + t) / (1.0 - t))
        psi = atanh * rsub

        def masked(a, p):
            rows = i * BI + jax.lax.broadcasted_iota(jnp.int32, (BI, BJ), 0)
            cols = j * BJ + jax.lax.broadcasted_iota(jnp.int32, (BI, BJ), 1)
            return jnp.where(rows == cols, 0.0, a * p)

        wpsi = jax.lax.cond(j == dchunk, masked, lambda a, p: a * p,
                            adjc, psi)
        wphi = wpsi * r
        a1v = a1v + wpsi
        degv = degv + adjc
        acc = acc + _dot(wphi, bc, ((1,), (0,)))
        return a1v, degv, acc

    zv = jnp.zeros((BI, BJ), jnp.float32)
    a1v, degv, acc = jax.lax.fori_loop(
        0, N // BJ, body, (zv, zv, jnp.zeros((BI, DA), jnp.float32)))

    a1 = jnp.sum(a1v, axis=1, keepdims=True)
    deg = jnp.sum(degv, axis=1, keepdims=True)
    m2 = acc[:, D:D + 1]
    a_coef = -(a1 + m2)
    degc = jnp.maximum(deg, 1e-8)
    betam = jnp.maximum(beta, EPS)
    mean_t = (betam / degc) * (a_coef * xi + acc[:, :D])
    v = _dot(mean_t, w_ref[...], ((1,), (0,))) + bias_ref[...]
    v_norm = jnp.sqrt(jnp.sum(v * v, axis=1, keepdims=True) + EPS)
    second = jnp.tanh(jnp.clip(v_norm / betam, -15.0, 15.0)) * v / v_norm
    b2 = jnp.sum(second * second, axis=1, keepdims=True)
    ab = jnp.sum(xi * second, axis=1, keepdims=True)
    num = (1.0 + 2.0 * ab + b2) * xi + beta * second
    dn = 1.0 + 2.0 * ab + pn2 * b2
    res = num / jnp.maximum(dn, EPS)
    rn = jnp.sqrt(jnp.sum(res * res, axis=1, keepdims=True) + EPS)
    res = jnp.where(rn > MAXNORM, res / rn * MAXNORM, res)
    xin = jnp.sqrt(pn2 + EPS)
    fb = jnp.where(xin > MAXNORM, xi / xin * MAXNORM, xi)
    out_ref[...] = jnp.where(deg > 0.5, res, fb)


def kernel(x, adj, weight, bias):
    qn2 = jnp.sum(x * x, axis=1, keepdims=True)
    invb = 1.0 / (1.0 - qn2)
    ones = jnp.ones((N, 1), jnp.float32)
    pad = jnp.zeros((N, DA - D - 2), jnp.float32)
    xa = jnp.concatenate([-2.0 * x * invb, invb, qn2 * invb, pad], axis=1)
    b = jnp.concatenate([x * invb, qn2 * invb, invb, pad], axis=1)
    return pl.pallas_call(
        _mmp_kernel,
        grid=(N // BI,),
        in_specs=[
            pl.BlockSpec((BI, D), lambda i: (i, 0)),
            pl.BlockSpec((BI, DA), lambda i: (i, 0)),
            pl.BlockSpec((N, DA), lambda i: (0, 0)),
            pl.BlockSpec((BI, N), lambda i: (i, 0)),
            pl.BlockSpec((D, D), lambda i: (0, 0)),
            pl.BlockSpec((1, D), lambda i: (0, 0)),
        ],
        out_specs=pl.BlockSpec((BI, D), lambda i: (i, 0)),
        out_shape=jax.ShapeDtypeStruct((N, D), jnp.float32),
    )(x, xa, b, adj, weight, bias.reshape(1, D))


# speedup vs baseline: 70.6447x; 1.4681x over previous
import jax
import jax.numpy as jnp
from jax.experimental import pallas as pl

N = 4096
D = 128
BI = 128
BJ = 512
DA = 136
EPS = 1e-15
MAXNORM = 1.0 - 1e-5
_HI = jax.lax.Precision.HIGHEST


def _dot(a, b, dims):
    return jax.lax.dot_general(a, b, (dims, ((), ())), precision=_HI,
                               preferred_element_type=jnp.float32)


def _mmp_kernel(xi_ref, xa_ref, b_ref, adj_ref, w_ref, bias_ref, out_ref):
    i = pl.program_id(0)
    xi = xi_ref[...]
    xa = xa_ref[...]
    pn2 = jnp.sum(xi * xi, axis=1, keepdims=True)
    beta = 1.0 - pn2
    dchunk = (i * BI) // BJ

    def body(j, carry):
        a1v, degv, acc = carry
        bc = b_ref[pl.ds(j * BJ, BJ), :]
        adjc = adj_ref[:, pl.ds(j * BJ, BJ)]
        ehat = jax.lax.dot_general(xa, bc, ((((1,), (1,))), ((), ())),
                                   preferred_element_type=jnp.float32)
        ehat = jnp.maximum(ehat, 0.0)
        r = 1.0 / (1.0 + ehat)
        ssq = (1.0 - r) + EPS
        rsub = jax.lax.rsqrt(ssq)
        subn = ssq * rsub
        t = jnp.minimum(subn, MAXNORM)
        atanh = 0.5 * jnp.log((1.0 + t) / (1.0 - t))
        psi = atanh * rsub

        def masked(a, p):
            rows = i * BI + jax.lax.broadcasted_iota(jnp.int32, (BI, BJ), 0)
            cols = j * BJ + jax.lax.broadcasted_iota(jnp.int32, (BI, BJ), 1)
            return jnp.where(rows == cols, 0.0, a * p)

        wpsi = jax.lax.cond(j == dchunk, masked, lambda a, p: a * p,
                            adjc, psi)
        wphi = wpsi * r
        a1v = a1v + wpsi
        degv = degv + adjc
        acc = acc + _dot(wphi, bc, ((1,), (0,)))
        return a1v, degv, acc

    zv = jnp.zeros((BI, BJ), jnp.float32)
    a1v, degv, acc = jax.lax.fori_loop(
        0, N // BJ, body, (zv, zv, jnp.zeros((BI, DA), jnp.float32)))

    a1 = jnp.sum(a1v, axis=1, keepdims=True)
    deg = jnp.sum(degv, axis=1, keepdims=True)
    m2 = acc[:, D:D + 1]
    a_coef = -(a1 + m2)
    degc = jnp.maximum(deg, 1e-8)
    betam = jnp.maximum(beta, EPS)
    mean_t = (betam / degc) * (a_coef * xi + acc[:, :D])
    v = _dot(mean_t, w_ref[...], ((1,), (0,))) + bias_ref[...]
    v_norm = jnp.sqrt(jnp.sum(v * v, axis=1, keepdims=True) + EPS)
    second = jnp.tanh(jnp.clip(v_norm / betam, -15.0, 15.0)) * v / v_norm
    b2 = jnp.sum(second * second, axis=1, keepdims=True)
    ab = jnp.sum(xi * second, axis=1, keepdims=True)
    num = (1.0 + 2.0 * ab + b2) * xi + beta * second
    dn = 1.0 + 2.0 * ab + pn2 * b2
    res = num / jnp.maximum(dn, EPS)
    rn = jnp.sqrt(jnp.sum(res * res, axis=1, keepdims=True) + EPS)
    res = jnp.where(rn > MAXNORM, res / rn * MAXNORM, res)
    xin = jnp.sqrt(pn2 + EPS)
    fb = jnp.where(xin > MAXNORM, xi / xin * MAXNORM, xi)
    out_ref[...] = jnp.where(deg > 0.5, res, fb)


def kernel(x, adj, weight, bias):
    qn2 = jnp.sum(x * x, axis=1, keepdims=True)
    invb = 1.0 / (1.0 - qn2)
    ones = jnp.ones((N, 1), jnp.float32)
    pad = jnp.zeros((N, DA - D - 2), jnp.float32)
    xa = jnp.concatenate([-2.0 * x * invb, invb, qn2 * invb, pad], axis=1)
    b = jnp.concatenate([x * invb, qn2 * invb, invb, pad], axis=1)
    return pl.pallas_call(
        _mmp_kernel,
        grid=(N // BI,),
        in_specs=[
            pl.BlockSpec((BI, D), lambda i: (i, 0)),
            pl.BlockSpec((BI, DA), lambda i: (i, 0)),
            pl.BlockSpec((N, DA), lambda i: (0, 0)),
            pl.BlockSpec((BI, N), lambda i: (i, 0)),
            pl.BlockSpec((D, D), lambda i: (0, 0)),
            pl.BlockSpec((1, D), lambda i: (0, 0)),
        ],
        out_specs=pl.BlockSpec((BI, D), lambda i: (i, 0)),
        out_shape=jax.ShapeDtypeStruct((N, D), jnp.float32),
    )(x, xa, b, adj, weight, bias.reshape(1, D))


# speedup vs baseline: 98.3791x; 1.3926x over previous
import jax
import jax.numpy as jnp
from jax.experimental import pallas as pl

N = 4096
D = 128
BI = 128
BJ = 512
DA = 136
EPS = 1e-15
MAXNORM = 1.0 - 1e-5
_HI = jax.lax.Precision.HIGHEST


def _dot(a, b, dims):
    return jax.lax.dot_general(a, b, (dims, ((), ())), precision=_HI,
                               preferred_element_type=jnp.float32)


def _mmp_kernel(xi_ref, xa_ref, b_ref, adj_ref, w_ref, bias_ref, out_ref):
    i = pl.program_id(0)
    xi = xi_ref[...]
    xa = xa_ref[...]
    pn2 = jnp.sum(xi * xi, axis=1, keepdims=True)
    beta = 1.0 - pn2
    dchunk = (i * BI) // BJ

    def body(j, carry):
        a1v, degv, acc = carry
        bc = b_ref[pl.ds(j * BJ, BJ), :]
        adjc = adj_ref[:, pl.ds(j * BJ, BJ)]
        ehat = jax.lax.dot_general(xa, bc, ((((1,), (1,))), ((), ())),
                                   preferred_element_type=jnp.float32)
        ehat = jnp.maximum(ehat, 0.0)
        r = 1.0 / (1.0 + ehat)
        ssq = (1.0 - r) + EPS
        rsub = jax.lax.rsqrt(ssq)
        subn = ssq * rsub
        t = jnp.minimum(subn, MAXNORM)
        atanh = 0.5 * jnp.log((1.0 + t) / (1.0 - t))
        psi = atanh * rsub

        def masked(a, p):
            rows = i * BI + jax.lax.broadcasted_iota(jnp.int32, (BI, BJ), 0)
            cols = j * BJ + jax.lax.broadcasted_iota(jnp.int32, (BI, BJ), 1)
            return jnp.where(rows == cols, 0.0, a * p)

        wpsi = jax.lax.cond(j == dchunk, masked, lambda a, p: a * p,
                            adjc, psi)
        wphi = wpsi * r
        a1v = a1v + wpsi
        degv = degv + adjc
        acc = acc + jax.lax.dot_general(wphi, bc, (((1,), (0,)), ((), ())),
                                        preferred_element_type=jnp.float32)
        return a1v, degv, acc

    zv = jnp.zeros((BI, BJ), jnp.float32)
    a1v, degv, acc = jax.lax.fori_loop(
        0, N // BJ, body, (zv, zv, jnp.zeros((BI, DA), jnp.float32)))

    a1 = jnp.sum(a1v, axis=1, keepdims=True)
    deg = jnp.sum(degv, axis=1, keepdims=True)
    m2 = acc[:, D:D + 1]
    a_coef = -(a1 + m2)
    degc = jnp.maximum(deg, 1e-8)
    betam = jnp.maximum(beta, EPS)
    mean_t = (betam / degc) * (a_coef * xi + acc[:, :D])
    v = _dot(mean_t, w_ref[...], ((1,), (0,))) + bias_ref[...]
    v_norm = jnp.sqrt(jnp.sum(v * v, axis=1, keepdims=True) + EPS)
    second = jnp.tanh(jnp.clip(v_norm / betam, -15.0, 15.0)) * v / v_norm
    b2 = jnp.sum(second * second, axis=1, keepdims=True)
    ab = jnp.sum(xi * second, axis=1, keepdims=True)
    num = (1.0 + 2.0 * ab + b2) * xi + beta * second
    dn = 1.0 + 2.0 * ab + pn2 * b2
    res = num / jnp.maximum(dn, EPS)
    rn = jnp.sqrt(jnp.sum(res * res, axis=1, keepdims=True) + EPS)
    res = jnp.where(rn > MAXNORM, res / rn * MAXNORM, res)
    xin = jnp.sqrt(pn2 + EPS)
    fb = jnp.where(xin > MAXNORM, xi / xin * MAXNORM, xi)
    out_ref[...] = jnp.where(deg > 0.5, res, fb)


def kernel(x, adj, weight, bias):
    qn2 = jnp.sum(x * x, axis=1, keepdims=True)
    invb = 1.0 / (1.0 - qn2)
    ones = jnp.ones((N, 1), jnp.float32)
    pad = jnp.zeros((N, DA - D - 2), jnp.float32)
    xa = jnp.concatenate([-2.0 * x * invb, invb, qn2 * invb, pad], axis=1)
    b = jnp.concatenate([x * invb, qn2 * invb, invb, pad], axis=1)
    return pl.pallas_call(
        _mmp_kernel,
        grid=(N // BI,),
        in_specs=[
            pl.BlockSpec((BI, D), lambda i: (i, 0)),
            pl.BlockSpec((BI, DA), lambda i: (i, 0)),
            pl.BlockSpec((N, DA), lambda i: (0, 0)),
            pl.BlockSpec((BI, N), lambda i: (i, 0)),
            pl.BlockSpec((D, D), lambda i: (0, 0)),
            pl.BlockSpec((1, D), lambda i: (0, 0)),
        ],
        out_specs=pl.BlockSpec((BI, D), lambda i: (i, 0)),
        out_shape=jax.ShapeDtypeStruct((N, D), jnp.float32),
    )(x, xa, b, adj, weight, bias.reshape(1, D))


# speedup vs baseline: 112.7713x; 1.1463x over previous
import math

import jax
import jax.numpy as jnp
from jax.experimental import pallas as pl

N = 4096
D = 128
BI = 128
BJ = 512
DA = 136
EPS = 1e-15
MAXNORM = 1.0 - 1e-5
ATANH_MAX = 0.5 * math.log((1.0 + MAXNORM) / (1.0 - MAXNORM))
_HI = jax.lax.Precision.HIGHEST


def _mmp_kernel(xi_ref, xa_ref, bt_ref, adj_ref, w_ref, bias_ref, out_ref):
    i = pl.program_id(0)
    xi = xi_ref[...]
    xa = xa_ref[...]
    pn2 = jnp.sum(xi * xi, axis=1, keepdims=True)
    beta = 1.0 - pn2
    dchunk = (i * BI) // BJ

    def fold(v):
        return sum(v[:, k * D:(k + 1) * D] for k in range(BJ // D))

    def body(j, carry):
        a1f, degf, acc = carry
        btc = bt_ref[:, pl.ds(j * BJ, BJ)]
        adjc = adj_ref[:, pl.ds(j * BJ, BJ)]
        e1 = 1.0 + jnp.maximum(
            jax.lax.dot_general(xa, btc, (((1,), (0,)), ((), ())),
                                preferred_element_type=jnp.float32), 0.0)
        r = 1.0 / e1
        ssq = (1.0 - r) + EPS
        rsub = jax.lax.rsqrt(ssq)
        subn = ssq * rsub
        u = 1.0 + subn
        atanh = jnp.where(subn < MAXNORM,
                          0.5 * jnp.log((u * u) * e1), ATANH_MAX)
        psi = atanh * rsub

        def masked(a, p):
            rows = i * BI + jax.lax.broadcasted_iota(jnp.int32, (BI, BJ), 0)
            cols = j * BJ + jax.lax.broadcasted_iota(jnp.int32, (BI, BJ), 1)
            return jnp.where(rows == cols, 0.0, a * p)

        wpsi = jax.lax.cond(j == dchunk, masked, lambda a, p: a * p,
                            adjc, psi)
        wphi = wpsi * r
        a1f = a1f + fold(wpsi)
        degf = degf + fold(adjc)
        acc = acc + jax.lax.dot_general(wphi, btc, (((1,), (1,)), ((), ())),
                                        preferred_element_type=jnp.float32)
        return a1f, degf, acc

    zf = jnp.zeros((BI, D), jnp.float32)
    a1f, degf, acc = jax.lax.fori_loop(
        0, N // BJ, body, (zf, zf, jnp.zeros((BI, DA), jnp.float32)))

    a1 = jnp.sum(a1f, axis=1, keepdims=True)
    deg = jnp.sum(degf, axis=1, keepdims=True)
    m2 = acc[:, D:D + 1]
    a_coef = -(a1 + m2)
    degc = jnp.maximum(deg, 1e-8)
    betam = jnp.maximum(beta, EPS)
    mean_t = (betam / degc) * (a_coef * xi + acc[:, :D])
    v = jax.lax.dot_general(mean_t, w_ref[...], (((1,), (0,)), ((), ())),
                            precision=_HI,
                            preferred_element_type=jnp.float32) + bias_ref[...]
    v_norm = jnp.sqrt(jnp.sum(v * v, axis=1, keepdims=True) + EPS)
    second = jnp.tanh(jnp.clip(v_norm / betam, -15.0, 15.0)) * v / v_norm
    b2 = jnp.sum(second * second, axis=1, keepdims=True)
    ab = jnp.sum(xi * second, axis=1, keepdims=True)
    num = (1.0 + 2.0 * ab + b2) * xi + beta * second
    dn = 1.0 + 2.0 * ab + pn2 * b2
    res = num / jnp.maximum(dn, EPS)
    rn = jnp.sqrt(jnp.sum(res * res, axis=1, keepdims=True) + EPS)
    res = jnp.where(rn > MAXNORM, res / rn * MAXNORM, res)
    xin = jnp.sqrt(pn2 + EPS)
    fb = jnp.where(xin > MAXNORM, xi / xin * MAXNORM, xi)
    out_ref[...] = jnp.where(deg > 0.5, res, fb)


def kernel(x, adj, weight, bias):
    qn2 = jnp.sum(x * x, axis=1, keepdims=True)
    invb = 1.0 / (1.0 - qn2)
    pad = jnp.zeros((N, DA - D - 2), jnp.float32)
    xa = jnp.concatenate([-2.0 * x * invb, invb, qn2 * invb, pad], axis=1)
    bt = jnp.concatenate([x * invb, qn2 * invb, invb, pad], axis=1).T
    return pl.pallas_call(
        _mmp_kernel,
        grid=(N // BI,),
        in_specs=[
            pl.BlockSpec((BI, D), lambda i: (i, 0)),
            pl.BlockSpec((BI, DA), lambda i: (i, 0)),
            pl.BlockSpec((DA, N), lambda i: (0, 0)),
            pl.BlockSpec((BI, N), lambda i: (i, 0)),
            pl.BlockSpec((D, D), lambda i: (0, 0)),
            pl.BlockSpec((1, D), lambda i: (0, 0)),
        ],
        out_specs=pl.BlockSpec((BI, D), lambda i: (i, 0)),
        out_shape=jax.ShapeDtypeStruct((N, D), jnp.float32),
    )(x, xa, bt, adj, weight, bias.reshape(1, D))


# speedup vs baseline: 134.2822x; 1.1907x over previous
import math

import jax
import jax.numpy as jnp
from jax.experimental import pallas as pl

N = 4096
D = 128
BI = 128
BJ = 1024
DA = 136
EPS = 1e-15
MAXNORM = 1.0 - 1e-5
ATANH_MAX = 0.5 * math.log((1.0 + MAXNORM) / (1.0 - MAXNORM))
_HI = jax.lax.Precision.HIGHEST


def _mmp_kernel(xi_ref, xa_ref, bt_ref, adj_ref, w_ref, bias_ref, out_ref):
    i = pl.program_id(0)
    xi = xi_ref[...]
    xa = xa_ref[...]
    pn2 = jnp.sum(xi * xi, axis=1, keepdims=True)
    beta = 1.0 - pn2
    dchunk = (i * BI) // BJ

    def fold(v):
        return sum(v[:, k * D:(k + 1) * D] for k in range(BJ // D))

    def body(j, carry):
        a1f, degf, acc = carry
        btc = bt_ref[:, pl.ds(j * BJ, BJ)]
        adjc = adj_ref[:, pl.ds(j * BJ, BJ)]
        e1 = 1.0 + jnp.maximum(
            jax.lax.dot_general(xa, btc, (((1,), (0,)), ((), ())),
                                preferred_element_type=jnp.float32), 0.0)
        r = 1.0 / e1
        ssq = (1.0 - r) + EPS
        rsub = jax.lax.rsqrt(ssq)
        subn = ssq * rsub
        u = 1.0 + subn
        atanh = jnp.where(subn < MAXNORM,
                          0.5 * jnp.log((u * u) * e1), ATANH_MAX)
        psi = atanh * rsub

        def masked(a, p):
            rows = i * BI + jax.lax.broadcasted_iota(jnp.int32, (BI, BJ), 0)
            cols = j * BJ + jax.lax.broadcasted_iota(jnp.int32, (BI, BJ), 1)
            return jnp.where(rows == cols, 0.0, a * p)

        wpsi = jax.lax.cond(j == dchunk, masked, lambda a, p: a * p,
                            adjc, psi)
        wphi = wpsi * r
        a1f = a1f + fold(wpsi)
        degf = degf + fold(adjc)
        acc = acc + jax.lax.dot_general(wphi, btc, (((1,), (1,)), ((), ())),
                                        preferred_element_type=jnp.float32)
        return a1f, degf, acc

    zf = jnp.zeros((BI, D), jnp.float32)
    a1f, degf, acc = jax.lax.fori_loop(
        0, N // BJ, body, (zf, zf, jnp.zeros((BI, DA), jnp.float32)))

    a1 = jnp.sum(a1f, axis=1, keepdims=True)
    deg = jnp.sum(degf, axis=1, keepdims=True)
    m2 = acc[:, D:D + 1]
    a_coef = -(a1 + m2)
    degc = jnp.maximum(deg, 1e-8)
    betam = jnp.maximum(beta, EPS)
    mean_t = (betam / degc) * (a_coef * xi + acc[:, :D])
    v = jax.lax.dot_general(mean_t, w_ref[...], (((1,), (0,)), ((), ())),
                            precision=_HI,
                            preferred_element_type=jnp.float32) + bias_ref[...]
    v_norm = jnp.sqrt(jnp.sum(v * v, axis=1, keepdims=True) + EPS)
    second = jnp.tanh(jnp.clip(v_norm / betam, -15.0, 15.0)) * v / v_norm
    b2 = jnp.sum(second * second, axis=1, keepdims=True)
    ab = jnp.sum(xi * second, axis=1, keepdims=True)
    num = (1.0 + 2.0 * ab + b2) * xi + beta * second
    dn = 1.0 + 2.0 * ab + pn2 * b2
    res = num / jnp.maximum(dn, EPS)
    rn = jnp.sqrt(jnp.sum(res * res, axis=1, keepdims=True) + EPS)
    res = jnp.where(rn > MAXNORM, res / rn * MAXNORM, res)
    xin = jnp.sqrt(pn2 + EPS)
    fb = jnp.where(xin > MAXNORM, xi / xin * MAXNORM, xi)
    out_ref[...] = jnp.where(deg > 0.5, res, fb)


def kernel(x, adj, weight, bias):
    qn2 = jnp.sum(x * x, axis=1, keepdims=True)
    invb = 1.0 / (1.0 - qn2)
    pad = jnp.zeros((N, DA - D - 2), jnp.float32)
    xa = jnp.concatenate([-2.0 * x * invb, invb, qn2 * invb, pad], axis=1)
    bt = jnp.concatenate([x * invb, qn2 * invb, invb, pad], axis=1).T
    return pl.pallas_call(
        _mmp_kernel,
        grid=(N // BI,),
        in_specs=[
            pl.BlockSpec((BI, D), lambda i: (i, 0)),
            pl.BlockSpec((BI, DA), lambda i: (i, 0)),
            pl.BlockSpec((DA, N), lambda i: (0, 0)),
            pl.BlockSpec((BI, N), lambda i: (i, 0)),
            pl.BlockSpec((D, D), lambda i: (0, 0)),
            pl.BlockSpec((1, D), lambda i: (0, 0)),
        ],
        out_specs=pl.BlockSpec((BI, D), lambda i: (i, 0)),
        out_shape=jax.ShapeDtypeStruct((N, D), jnp.float32),
    )(x, xa, bt, adj, weight, bias.reshape(1, D))


# speedup vs baseline: 158.3296x; 1.1791x over previous
import math

import jax
import jax.numpy as jnp
from jax.experimental import pallas as pl

N = 4096
D = 128
BI = 128
BJ = 4096
DA = 136
EPS = 1e-15
MAXNORM = 1.0 - 1e-5
ATANH_MAX = 0.5 * math.log((1.0 + MAXNORM) / (1.0 - MAXNORM))
_HI = jax.lax.Precision.HIGHEST


def _mmp_kernel(xi_ref, xa_ref, bt_ref, adj_ref, w_ref, bias_ref, out_ref):
    i = pl.program_id(0)
    xi = xi_ref[...]
    xa = xa_ref[...]
    pn2 = jnp.sum(xi * xi, axis=1, keepdims=True)
    beta = 1.0 - pn2
    dchunk = (i * BI) // BJ

    def fold(v):
        return sum(v[:, k * D:(k + 1) * D] for k in range(BJ // D))

    def body(j, carry):
        a1f, degf, acc = carry
        btc = bt_ref[:, pl.ds(j * BJ, BJ)]
        adjc = adj_ref[:, pl.ds(j * BJ, BJ)]
        e1 = 1.0 + jnp.maximum(
            jax.lax.dot_general(xa, btc, (((1,), (0,)), ((), ())),
                                preferred_element_type=jnp.float32), 0.0)
        r = 1.0 / e1
        ssq = (1.0 - r) + EPS
        rsub = jax.lax.rsqrt(ssq)
        subn = ssq * rsub
        u = 1.0 + subn
        atanh = jnp.where(subn < MAXNORM,
                          0.5 * jnp.log((u * u) * e1), ATANH_MAX)
        psi = atanh * rsub

        def masked(a, p):
            rows = i * BI + jax.lax.broadcasted_iota(jnp.int32, (BI, BJ), 0)
            cols = j * BJ + jax.lax.broadcasted_iota(jnp.int32, (BI, BJ), 1)
            return jnp.where(rows == cols, 0.0, a * p)

        wpsi = jax.lax.cond(j == dchunk, masked, lambda a, p: a * p,
                            adjc, psi)
        wphi = wpsi * r
        a1f = a1f + fold(wpsi)
        degf = degf + fold(adjc)
        acc = acc + jax.lax.dot_general(wphi, btc, (((1,), (1,)), ((), ())),
                                        preferred_element_type=jnp.float32)
        return a1f, degf, acc

    zf = jnp.zeros((BI, D), jnp.float32)
    a1f, degf, acc = jax.lax.fori_loop(
        0, N // BJ, body, (zf, zf, jnp.zeros((BI, DA), jnp.float32)))

    a1 = jnp.sum(a1f, axis=1, keepdims=True)
    deg = jnp.sum(degf, axis=1, keepdims=True)
    m2 = acc[:, D:D + 1]
    a_coef = -(a1 + m2)
    degc = jnp.maximum(deg, 1e-8)
    betam = jnp.maximum(beta, EPS)
    mean_t = (betam / degc) * (a_coef * xi + acc[:, :D])
    v = jax.lax.dot_general(mean_t, w_ref[...], (((1,), (0,)), ((), ())),
                            precision=_HI,
                            preferred_element_type=jnp.float32) + bias_ref[...]
    v_norm = jnp.sqrt(jnp.sum(v * v, axis=1, keepdims=True) + EPS)
    second = jnp.tanh(jnp.clip(v_norm / betam, -15.0, 15.0)) * v / v_norm
    b2 = jnp.sum(second * second, axis=1, keepdims=True)
    ab = jnp.sum(xi * second, axis=1, keepdims=True)
    num = (1.0 + 2.0 * ab + b2) * xi + beta * second
    dn = 1.0 + 2.0 * ab + pn2 * b2
    res = num / jnp.maximum(dn, EPS)
    rn = jnp.sqrt(jnp.sum(res * res, axis=1, keepdims=True) + EPS)
    res = jnp.where(rn > MAXNORM, res / rn * MAXNORM, res)
    xin = jnp.sqrt(pn2 + EPS)
    fb = jnp.where(xin > MAXNORM, xi / xin * MAXNORM, xi)
    out_ref[...] = jnp.where(deg > 0.5, res, fb)


def kernel(x, adj, weight, bias):
    qn2 = jnp.sum(x * x, axis=1, keepdims=True)
    invb = 1.0 / (1.0 - qn2)
    pad = jnp.zeros((N, DA - D - 2), jnp.float32)
    xa = jnp.concatenate([-2.0 * x * invb, invb, qn2 * invb, pad], axis=1)
    bt = jnp.concatenate([x * invb, qn2 * invb, invb, pad], axis=1).T
    return pl.pallas_call(
        _mmp_kernel,
        grid=(N // BI,),
        in_specs=[
            pl.BlockSpec((BI, D), lambda i: (i, 0)),
            pl.BlockSpec((BI, DA), lambda i: (i, 0)),
            pl.BlockSpec((DA, N), lambda i: (0, 0)),
            pl.BlockSpec((BI, N), lambda i: (i, 0)),
            pl.BlockSpec((D, D), lambda i: (0, 0)),
            pl.BlockSpec((1, D), lambda i: (0, 0)),
        ],
        out_specs=pl.BlockSpec((BI, D), lambda i: (i, 0)),
        out_shape=jax.ShapeDtypeStruct((N, D), jnp.float32),
    )(x, xa, bt, adj, weight, bias.reshape(1, D))


# speedup vs baseline: 165.1620x; 1.0432x over previous
import math

import jax
import jax.numpy as jnp
from jax.experimental import pallas as pl

N = 4096
D = 128
BI = 256
BJ = 4096
DA = 136
EPS = 1e-15
MAXNORM = 1.0 - 1e-5
ATANH_MAX = 0.5 * math.log((1.0 + MAXNORM) / (1.0 - MAXNORM))
_HI = jax.lax.Precision.HIGHEST


def _mmp_kernel(xi_ref, xa_ref, bt_ref, adj_ref, w_ref, bias_ref, out_ref):
    i = pl.program_id(0)
    xi = xi_ref[...]
    xa = xa_ref[...]
    pn2 = jnp.sum(xi * xi, axis=1, keepdims=True)
    beta = 1.0 - pn2
    dchunk = (i * BI) // BJ

    def fold(v):
        return sum(v[:, k * D:(k + 1) * D] for k in range(BJ // D))

    def body(j, carry):
        a1f, degf, acc = carry
        btc = bt_ref[:, pl.ds(j * BJ, BJ)]
        adjc = adj_ref[:, pl.ds(j * BJ, BJ)]
        e1 = 1.0 + jnp.maximum(
            jax.lax.dot_general(xa, btc, (((1,), (0,)), ((), ())),
                                preferred_element_type=jnp.float32), 0.0)
        r = 1.0 / e1
        ssq = (1.0 - r) + EPS
        rsub = jax.lax.rsqrt(ssq)
        subn = ssq * rsub
        u = 1.0 + subn
        atanh = jnp.where(subn < MAXNORM,
                          0.5 * jnp.log((u * u) * e1), ATANH_MAX)
        psi = atanh * rsub

        def masked(a, p):
            rows = i * BI + jax.lax.broadcasted_iota(jnp.int32, (BI, BJ), 0)
            cols = j * BJ + jax.lax.broadcasted_iota(jnp.int32, (BI, BJ), 1)
            return jnp.where(rows == cols, 0.0, a * p)

        wpsi = jax.lax.cond(j == dchunk, masked, lambda a, p: a * p,
                            adjc, psi)
        wphi = wpsi * r
        a1f = a1f + fold(wpsi)
        degf = degf + fold(adjc)
        acc = acc + jax.lax.dot_general(wphi, btc, (((1,), (1,)), ((), ())),
                                        preferred_element_type=jnp.float32)
        return a1f, degf, acc

    zf = jnp.zeros((BI, D), jnp.float32)
    a1f, degf, acc = jax.lax.fori_loop(
        0, N // BJ, body, (zf, zf, jnp.zeros((BI, DA), jnp.float32)))

    a1 = jnp.sum(a1f, axis=1, keepdims=True)
    deg = jnp.sum(degf, axis=1, keepdims=True)
    m2 = acc[:, D:D + 1]
    a_coef = -(a1 + m2)
    degc = jnp.maximum(deg, 1e-8)
    betam = jnp.maximum(beta, EPS)
    mean_t = (betam / degc) * (a_coef * xi + acc[:, :D])
    v = jax.lax.dot_general(mean_t, w_ref[...], (((1,), (0,)), ((), ())),
                            precision=_HI,
                            preferred_element_type=jnp.float32) + bias_ref[...]
    v_norm = jnp.sqrt(jnp.sum(v * v, axis=1, keepdims=True) + EPS)
    second = jnp.tanh(jnp.clip(v_norm / betam, -15.0, 15.0)) * v / v_norm
    b2 = jnp.sum(second * second, axis=1, keepdims=True)
    ab = jnp.sum(xi * second, axis=1, keepdims=True)
    num = (1.0 + 2.0 * ab + b2) * xi + beta * second
    dn = 1.0 + 2.0 * ab + pn2 * b2
    res = num / jnp.maximum(dn, EPS)
    rn = jnp.sqrt(jnp.sum(res * res, axis=1, keepdims=True) + EPS)
    res = jnp.where(rn > MAXNORM, res / rn * MAXNORM, res)
    xin = jnp.sqrt(pn2 + EPS)
    fb = jnp.where(xin > MAXNORM, xi / xin * MAXNORM, xi)
    out_ref[...] = jnp.where(deg > 0.5, res, fb)


def kernel(x, adj, weight, bias):
    qn2 = jnp.sum(x * x, axis=1, keepdims=True)
    invb = 1.0 / (1.0 - qn2)
    pad = jnp.zeros((N, DA - D - 2), jnp.float32)
    xa = jnp.concatenate([-2.0 * x * invb, invb, qn2 * invb, pad], axis=1)
    bt = jnp.concatenate([x * invb, qn2 * invb, invb, pad], axis=1).T
    return pl.pallas_call(
        _mmp_kernel,
        grid=(N // BI,),
        in_specs=[
            pl.BlockSpec((BI, D), lambda i: (i, 0)),
            pl.BlockSpec((BI, DA), lambda i: (i, 0)),
            pl.BlockSpec((DA, N), lambda i: (0, 0)),
            pl.BlockSpec((BI, N), lambda i: (i, 0)),
            pl.BlockSpec((D, D), lambda i: (0, 0)),
            pl.BlockSpec((1, D), lambda i: (0, 0)),
        ],
        out_specs=pl.BlockSpec((BI, D), lambda i: (i, 0)),
        out_shape=jax.ShapeDtypeStruct((N, D), jnp.float32),
    )(x, xa, bt, adj, weight, bias.reshape(1, D))


# speedup vs baseline: 167.6506x; 1.0151x over previous
import math

import jax
import jax.numpy as jnp
from jax.experimental import pallas as pl

N = 4096
D = 128
BI = 512
BJ = 4096
DA = 136
EPS = 1e-15
MAXNORM = 1.0 - 1e-5
ATANH_MAX = 0.5 * math.log((1.0 + MAXNORM) / (1.0 - MAXNORM))
_HI = jax.lax.Precision.HIGHEST


def _mmp_kernel(xi_ref, xa_ref, bt_ref, adj_ref, w_ref, bias_ref, out_ref):
    i = pl.program_id(0)
    xi = xi_ref[...]
    xa = xa_ref[...]
    pn2 = jnp.sum(xi * xi, axis=1, keepdims=True)
    beta = 1.0 - pn2
    dchunk = (i * BI) // BJ

    def fold(v):
        return sum(v[:, k * D:(k + 1) * D] for k in range(BJ // D))

    def body(j, carry):
        a1f, degf, acc = carry
        btc = bt_ref[:, pl.ds(j * BJ, BJ)]
        adjc = adj_ref[:, pl.ds(j * BJ, BJ)]
        e1 = 1.0 + jnp.maximum(
            jax.lax.dot_general(xa, btc, (((1,), (0,)), ((), ())),
                                preferred_element_type=jnp.float32), 0.0)
        r = 1.0 / e1
        ssq = (1.0 - r) + EPS
        rsub = jax.lax.rsqrt(ssq)
        subn = ssq * rsub
        u = 1.0 + subn
        atanh = jnp.where(subn < MAXNORM,
                          0.5 * jnp.log((u * u) * e1), ATANH_MAX)
        psi = atanh * rsub

        def masked(a, p):
            rows = i * BI + jax.lax.broadcasted_iota(jnp.int32, (BI, BJ), 0)
            cols = j * BJ + jax.lax.broadcasted_iota(jnp.int32, (BI, BJ), 1)
            return jnp.where(rows == cols, 0.0, a * p)

        wpsi = jax.lax.cond(j == dchunk, masked, lambda a, p: a * p,
                            adjc, psi)
        wphi = wpsi * r
        a1f = a1f + fold(wpsi)
        degf = degf + fold(adjc)
        acc = acc + jax.lax.dot_general(wphi, btc, (((1,), (1,)), ((), ())),
                                        preferred_element_type=jnp.float32)
        return a1f, degf, acc

    zf = jnp.zeros((BI, D), jnp.float32)
    a1f, degf, acc = jax.lax.fori_loop(
        0, N // BJ, body, (zf, zf, jnp.zeros((BI, DA), jnp.float32)))

    a1 = jnp.sum(a1f, axis=1, keepdims=True)
    deg = jnp.sum(degf, axis=1, keepdims=True)
    m2 = acc[:, D:D + 1]
    a_coef = -(a1 + m2)
    degc = jnp.maximum(deg, 1e-8)
    betam = jnp.maximum(beta, EPS)
    mean_t = (betam / degc) * (a_coef * xi + acc[:, :D])
    v = jax.lax.dot_general(mean_t, w_ref[...], (((1,), (0,)), ((), ())),
                            precision=_HI,
                            preferred_element_type=jnp.float32) + bias_ref[...]
    v_norm = jnp.sqrt(jnp.sum(v * v, axis=1, keepdims=True) + EPS)
    second = jnp.tanh(jnp.clip(v_norm / betam, -15.0, 15.0)) * v / v_norm
    b2 = jnp.sum(second * second, axis=1, keepdims=True)
    ab = jnp.sum(xi * second, axis=1, keepdims=True)
    num = (1.0 + 2.0 * ab + b2) * xi + beta * second
    dn = 1.0 + 2.0 * ab + pn2 * b2
    res = num / jnp.maximum(dn, EPS)
    rn = jnp.sqrt(jnp.sum(res * res, axis=1, keepdims=True) + EPS)
    res = jnp.where(rn > MAXNORM, res / rn * MAXNORM, res)
    xin = jnp.sqrt(pn2 + EPS)
    fb = jnp.where(xin > MAXNORM, xi / xin * MAXNORM, xi)
    out_ref[...] = jnp.where(deg > 0.5, res, fb)


def kernel(x, adj, weight, bias):
    qn2 = jnp.sum(x * x, axis=1, keepdims=True)
    invb = 1.0 / (1.0 - qn2)
    pad = jnp.zeros((N, DA - D - 2), jnp.float32)
    xa = jnp.concatenate([-2.0 * x * invb, invb, qn2 * invb, pad], axis=1)
    bt = jnp.concatenate([x * invb, qn2 * invb, invb, pad], axis=1).T
    return pl.pallas_call(
        _mmp_kernel,
        grid=(N // BI,),
        in_specs=[
            pl.BlockSpec((BI, D), lambda i: (i, 0)),
            pl.BlockSpec((BI, DA), lambda i: (i, 0)),
            pl.BlockSpec((DA, N), lambda i: (0, 0)),
            pl.BlockSpec((BI, N), lambda i: (i, 0)),
            pl.BlockSpec((D, D), lambda i: (0, 0)),
            pl.BlockSpec((1, D), lambda i: (0, 0)),
        ],
        out_specs=pl.BlockSpec((BI, D), lambda i: (i, 0)),
        out_shape=jax.ShapeDtypeStruct((N, D), jnp.float32),
    )(x, xa, bt, adj, weight, bias.reshape(1, D))
